# baseline (speedup 1.0000x reference)
import functools

import numpy as np

import jax
import jax.numpy as jnp
from jax import lax
from jax.experimental import pallas as pl
from jax.experimental.pallas import tpu as pltpu
from jax.experimental.pallas import tpu_sc as plsc

H = 8
D = 16
C = 128
CH = C // 2
OUT = 16
K = 16
L = 16
NC, NS = 2, 16
G = 8
JP = 10240
CHUNK = JP // NS
ROUNDS = CHUNK // G
NBUF = 4
BLK = 1024
SCALE = 1.0 / (D ** 0.5)
NEG_SLOPE = 0.2


def _qkv_body(x_ref, wqt_ref, whi_ref, wlo_ref, q_ref, kv_ref):
    xb = x_ref[...]
    q = jnp.dot(xb, wqt_ref[...], preferred_element_type=jnp.float32)
    hi = jnp.dot(xb, whi_ref[...], preferred_element_type=jnp.float32)
    lo = jnp.dot(xb, wlo_ref[...], preferred_element_type=jnp.float32)
    hi_i = lax.convert_element_type(
        lax.bitcast_convert_type(
            lax.convert_element_type(hi, jnp.bfloat16), jnp.int16),
        jnp.int32)
    lo_i = lax.convert_element_type(
        lax.bitcast_convert_type(
            lax.convert_element_type(lo, jnp.bfloat16), jnp.int16),
        jnp.int32)
    word = (hi_i << 16) | (lo_i & 0xFFFF)
    q_ref[0] = q[:, :CH]
    q_ref[1] = q[:, CH:]
    kv_ref[0] = word[:, :CH]
    kv_ref[1] = word[:, CH:]


def _proj_body(o_ref, wpt0_ref, wpt1_ref, y_ref):
    y_ref[...] = (
        jnp.dot(o_ref[0], wpt0_ref[...], preferred_element_type=jnp.float32)
        + jnp.dot(o_ref[1], wpt1_ref[...], preferred_element_type=jnp.float32))


def _sc_body(q_hbm, kv_hbm, idx_hbm, out_hbm, idx_v, kv_v2, q_v2, out_v2,
             kv_sh, isem, ssem, ksem0, ksem1, ksem2, ksem3,
             qsem0, qsem1, qsem2, qsem3, osem0, osem1, osem2, osem3):
    cid = lax.axis_index("c")
    sid = lax.axis_index("s")
    base0 = sid * CHUNK
    iota = lax.iota(jnp.int32, L)
    ksems = (ksem0, ksem1, ksem2, ksem3)
    qsems = (qsem0, qsem1, qsem2, qsem3)
    osems = (osem0, osem1, osem2, osem3)

    pltpu.async_copy(idx_hbm.at[sid], idx_v, isem)

    @pl.when(sid == 0)
    def _():
        pltpu.async_copy(kv_hbm.at[cid], kv_sh, ssem).wait()

    plsc.subcore_barrier()
    pltpu.make_async_copy(idx_hbm.at[sid], idx_v, isem).wait()

    def start_round(r, b):
        base = base0 + r * G
        pltpu.async_copy(q_hbm.at[cid, pl.ds(base, G)], q_v2.at[b], qsems[b])
        pltpu.async_copy(kv_sh.at[idx_v.at[r]], kv_v2.at[b], ksems[b])

    def wait_round(r, b):
        base = base0 + r * G
        pltpu.make_async_copy(q_hbm.at[cid, pl.ds(base, G)], q_v2.at[b],
                              qsems[b]).wait()
        pltpu.make_async_copy(kv_sh.at[idx_v.at[r]], kv_v2.at[b],
                              ksems[b]).wait()

    def compute_round(r, b):
        kv_v = kv_v2.at[b]
        q_all = q_v2.at[b]
        out_all = out_v2.at[b]
        base = base0 + r * G
        row0 = 0

        @plsc.parallel_loop(0, G, 1, unroll=4)
        def node_body(g):
            g16 = g * L
            row_idx = g16 + iota
            nrow = row0 + g
            mask_hi = jnp.int32(-65536)

            def softmax(acc):
                a = acc * SCALE
                a = jnp.where(a >= 0.0, a, NEG_SLOPE * a)
                m = jnp.max(a)
                e = jnp.exp(a - m)
                return e / jnp.sum(e)

            wvecs = [None] * 4
            for p in range(2):
                q_e = q_all[nrow, pl.ds((2 * p) * D, D)]
                q_o = q_all[nrow, pl.ds((2 * p + 1) * D, D)]
                pe = [jnp.zeros((L,), jnp.float32) for _ in range(2)]
                po = [jnp.zeros((L,), jnp.float32) for _ in range(2)]
                for d in range(D):
                    w_i = p * D + d
                    word = plsc.load_gather(
                        kv_v, [row_idx, jnp.full((L,), w_i, jnp.int32)])
                    fe = plsc.bitcast(word & mask_hi, jnp.float32)
                    fo = plsc.bitcast(word << 16, jnp.float32)
                    pe[d % 2] = pe[d % 2] + q_e[d] * fe
                    po[d % 2] = po[d % 2] + q_o[d] * fo
                wvecs[2 * p] = softmax(pe[0] + pe[1])
                wvecs[2 * p + 1] = softmax(po[0] + po[1])
            for p in range(2):
                we = wvecs[2 * p]
                wo = wvecs[2 * p + 1]
                pe = [jnp.zeros((L,), jnp.float32) for _ in range(2)]
                po = [jnp.zeros((L,), jnp.float32) for _ in range(2)]
                for t in range(L):
                    word = kv_v[g16 + t, pl.ds(CH // 2 + p * D, D)]
                    fe = plsc.bitcast(word & mask_hi, jnp.float32)
                    fo = plsc.bitcast(word << 16, jnp.float32)
                    pe[t % 2] = pe[t % 2] + we[t] * fe
                    po[t % 2] = po[t % 2] + wo[t] * fo
                out_all[nrow, pl.ds((2 * p) * D, D)] = pe[0] + pe[1]
                out_all[nrow, pl.ds((2 * p + 1) * D, D)] = po[0] + po[1]
        pltpu.async_copy(out_all, out_hbm.at[cid, pl.ds(base, G)], osems[b])

    for b in range(NBUF):
        start_round(b, b)

    def outer(p, carry):
        for b in range(NBUF):
            r = NBUF * p + b
            wait_round(r, b)

            @pl.when(p > 0)
            def _():
                pltpu.make_async_copy(
                    out_v2.at[b], out_hbm.at[cid, pl.ds(base0, G)],
                    osems[b]).wait()

            compute_round(r, b)

            @pl.when(r + NBUF < ROUNDS)
            def _():
                start_round(r + NBUF, b)
        return carry

    lax.fori_loop(0, ROUNDS // NBUF, outer, 0)
    for b in range(NBUF):
        pltpu.make_async_copy(out_v2.at[b], out_hbm.at[cid, pl.ds(base0, G)],
                              osems[b]).wait()


_sc_call = pl.kernel(
    _sc_body,
    out_type=jax.ShapeDtypeStruct((NC, JP, CH), jnp.float32),
    mesh=plsc.VectorSubcoreMesh(
        core_axis_name="c", subcore_axis_name="s",
        num_cores=NC, num_subcores=NS),
    scratch_types=[
        pltpu.VMEM((ROUNDS, G * K), jnp.int32),
        pltpu.VMEM((NBUF, G * K, CH), jnp.int32),
        pltpu.VMEM((NBUF, G, CH), jnp.float32),
        pltpu.VMEM((NBUF, G, CH), jnp.float32),
        pltpu.VMEM_SHARED((JP, CH), jnp.int32),
    ] + [pltpu.SemaphoreType.DMA] * (2 + 3 * NBUF),
    compiler_params=pltpu.CompilerParams(
        use_tc_tiling_on_sc=False, needs_layout_passes=False),
)

_w = np.arange(C)
_u = _w % CH
_is_v = (_u >= CH // 2).astype(np.int64)
_P = 2 * (_w // CH) + (_u % (CH // 2)) // D
_f_hi = 2 * _P * D + (_w % D) + C * _is_v
_f_lo = _f_hi + D


@jax.jit
def kernel(x, nbr_idx, Wq, Wk, Wv, Wp):
    B, J, Cin = x.shape
    x2 = x.reshape(J, Cin)
    idx = nbr_idx.reshape(J, K).astype(jnp.int32)
    x_pad = jnp.pad(x2, ((0, JP - J), (0, 0)))
    idx3 = jnp.pad(idx, ((0, JP - J), (0, 0))).reshape(NS, ROUNDS, G * K)

    Wc = jnp.concatenate([Wk.T, Wv.T], axis=1)
    W_hi = Wc[:, _f_hi]
    W_lo = Wc[:, _f_lo]

    q_pair, kv_pair = pl.pallas_call(
        _qkv_body,
        grid=(JP // BLK,),
        in_specs=[
            pl.BlockSpec((BLK, Cin), lambda i: (i, 0)),
            pl.BlockSpec((Cin, C), lambda i: (0, 0)),
            pl.BlockSpec((Cin, C), lambda i: (0, 0)),
            pl.BlockSpec((Cin, C), lambda i: (0, 0)),
        ],
        out_specs=[
            pl.BlockSpec((NC, BLK, CH), lambda i: (0, i, 0)),
            pl.BlockSpec((NC, BLK, CH), lambda i: (0, i, 0)),
        ],
        out_shape=[
            jax.ShapeDtypeStruct((NC, JP, CH), jnp.float32),
            jax.ShapeDtypeStruct((NC, JP, CH), jnp.int32),
        ],
    )(x_pad, Wq.T, W_hi, W_lo)

    out_pair = _sc_call(q_pair, kv_pair, idx3)

    WpT = Wp.T
    y_pad = pl.pallas_call(
        _proj_body,
        grid=(JP // BLK,),
        in_specs=[
            pl.BlockSpec((NC, BLK, CH), lambda i: (0, i, 0)),
            pl.BlockSpec((CH, OUT), lambda i: (0, 0)),
            pl.BlockSpec((CH, OUT), lambda i: (0, 0)),
        ],
        out_specs=pl.BlockSpec((BLK, OUT), lambda i: (i, 0)),
        out_shape=jax.ShapeDtypeStruct((JP, OUT), jnp.float32),
    )(out_pair, WpT[:CH], WpT[CH:])

    return y_pad[:J].reshape(B, J, OUT)


# speedup vs baseline: 1.0303x; 1.0303x over previous
import functools

import numpy as np

import jax
import jax.numpy as jnp
from jax import lax
from jax.experimental import pallas as pl
from jax.experimental.pallas import tpu as pltpu
from jax.experimental.pallas import tpu_sc as plsc

H = 8
D = 16
C = 128
CH = C // 2
OUT = 16
K = 16
L = 16
NC, NS = 2, 16
G = 8
GN = 16
JP = 10240
CHUNK = JP // NS
IROWS = CHUNK // G
ROUNDS = CHUNK // GN
NBUF = 4
BLK = 1024
SCALE = 1.0 / (D ** 0.5)
NEG_SLOPE = 0.2


def _qkv_body(x_ref, wqt_ref, whi_ref, wlo_ref, q_ref, kv_ref):
    xb = x_ref[...]
    q = jnp.dot(xb, wqt_ref[...], preferred_element_type=jnp.float32)
    hi = jnp.dot(xb, whi_ref[...], preferred_element_type=jnp.float32)
    lo = jnp.dot(xb, wlo_ref[...], preferred_element_type=jnp.float32)
    hi_i = lax.convert_element_type(
        lax.bitcast_convert_type(
            lax.convert_element_type(hi, jnp.bfloat16), jnp.int16),
        jnp.int32)
    lo_i = lax.convert_element_type(
        lax.bitcast_convert_type(
            lax.convert_element_type(lo, jnp.bfloat16), jnp.int16),
        jnp.int32)
    word = (hi_i << 16) | (lo_i & 0xFFFF)
    q_ref[0] = q[:, :CH]
    q_ref[1] = q[:, CH:]
    kv_ref[0] = word[:, :CH]
    kv_ref[1] = word[:, CH:]


def _proj_body(o_ref, wpt0_ref, wpt1_ref, y_ref):
    y_ref[...] = (
        jnp.dot(o_ref[0], wpt0_ref[...], preferred_element_type=jnp.float32)
        + jnp.dot(o_ref[1], wpt1_ref[...], preferred_element_type=jnp.float32))


def _sc_body(q_hbm, kv_hbm, idx_hbm, out_hbm, idx_v, kv_v2, q_v2, out_v2,
             kv_sh, isem, ssem, ksem0, ksem1, ksem2, ksem3,
             qsem0, qsem1, qsem2, qsem3, osem0, osem1, osem2, osem3):
    cid = lax.axis_index("c")
    sid = lax.axis_index("s")
    base0 = sid * CHUNK
    iota = lax.iota(jnp.int32, L)
    ksems = (ksem0, ksem1, ksem2, ksem3)
    qsems = (qsem0, qsem1, qsem2, qsem3)
    osems = (osem0, osem1, osem2, osem3)

    pltpu.async_copy(idx_hbm.at[sid], idx_v, isem)

    @pl.when(sid == 0)
    def _():
        pltpu.async_copy(kv_hbm.at[cid], kv_sh, ssem).wait()

    plsc.subcore_barrier()
    pltpu.make_async_copy(idx_hbm.at[sid], idx_v, isem).wait()

    def start_round(r, b):
        base = base0 + r * GN
        pltpu.async_copy(q_hbm.at[cid, pl.ds(base, GN)], q_v2.at[b], qsems[b])
        pltpu.async_copy(kv_sh.at[idx_v.at[2 * r]],
                         kv_v2.at[b, pl.ds(0, G * K)], ksems[b])
        pltpu.async_copy(kv_sh.at[idx_v.at[2 * r + 1]],
                         kv_v2.at[b, pl.ds(G * K, G * K)], ksems[b])

    def wait_round(r, b):
        base = base0 + r * GN
        pltpu.make_async_copy(q_hbm.at[cid, pl.ds(base, GN)], q_v2.at[b],
                              qsems[b]).wait()
        pltpu.make_async_copy(kv_sh.at[idx_v.at[2 * r]],
                              kv_v2.at[b, pl.ds(0, G * K)], ksems[b]).wait()
        pltpu.make_async_copy(kv_sh.at[idx_v.at[2 * r + 1]],
                              kv_v2.at[b, pl.ds(G * K, G * K)],
                              ksems[b]).wait()

    def compute_round(r, b):
        kv_v = kv_v2.at[b]
        q_all = q_v2.at[b]
        out_all = out_v2.at[b]
        base = base0 + r * GN
        row0 = 0

        @plsc.parallel_loop(0, GN, 1, unroll=2)
        def node_body(g):
            g16 = g * L
            row_idx = g16 + iota
            nrow = row0 + g
            mask_hi = jnp.int32(-65536)

            def softmax(acc):
                a = acc * SCALE
                a = jnp.where(a >= 0.0, a, NEG_SLOPE * a)
                m = jnp.max(a)
                e = jnp.exp(a - m)
                return e / jnp.sum(e)

            wvecs = [None] * 4
            for p in range(2):
                q_e = q_all[nrow, pl.ds((2 * p) * D, D)]
                q_o = q_all[nrow, pl.ds((2 * p + 1) * D, D)]
                pe = [jnp.zeros((L,), jnp.float32) for _ in range(2)]
                po = [jnp.zeros((L,), jnp.float32) for _ in range(2)]
                for d in range(D):
                    w_i = p * D + d
                    word = plsc.load_gather(
                        kv_v, [row_idx, jnp.full((L,), w_i, jnp.int32)])
                    fe = plsc.bitcast(word & mask_hi, jnp.float32)
                    fo = plsc.bitcast(word << 16, jnp.float32)
                    pe[d % 2] = pe[d % 2] + q_e[d] * fe
                    po[d % 2] = po[d % 2] + q_o[d] * fo
                wvecs[2 * p] = softmax(pe[0] + pe[1])
                wvecs[2 * p + 1] = softmax(po[0] + po[1])
            for p in range(2):
                we = wvecs[2 * p]
                wo = wvecs[2 * p + 1]
                pe = [jnp.zeros((L,), jnp.float32) for _ in range(2)]
                po = [jnp.zeros((L,), jnp.float32) for _ in range(2)]
                for t in range(L):
                    word = kv_v[g16 + t, pl.ds(CH // 2 + p * D, D)]
                    fe = plsc.bitcast(word & mask_hi, jnp.float32)
                    fo = plsc.bitcast(word << 16, jnp.float32)
                    pe[t % 2] = pe[t % 2] + we[t] * fe
                    po[t % 2] = po[t % 2] + wo[t] * fo
                out_all[nrow, pl.ds((2 * p) * D, D)] = pe[0] + pe[1]
                out_all[nrow, pl.ds((2 * p + 1) * D, D)] = po[0] + po[1]
        pltpu.async_copy(out_all, out_hbm.at[cid, pl.ds(base, GN)], osems[b])

    for b in range(NBUF):
        start_round(b, b)

    def outer(p, carry):
        for b in range(NBUF):
            r = NBUF * p + b
            wait_round(r, b)

            @pl.when(p > 0)
            def _():
                pltpu.make_async_copy(
                    out_v2.at[b], out_hbm.at[cid, pl.ds(base0, GN)],
                    osems[b]).wait()

            compute_round(r, b)

            @pl.when(r + NBUF < ROUNDS)
            def _():
                start_round(r + NBUF, b)
        return carry

    lax.fori_loop(0, ROUNDS // NBUF, outer, 0)
    for b in range(NBUF):
        pltpu.make_async_copy(out_v2.at[b], out_hbm.at[cid, pl.ds(base0, GN)],
                              osems[b]).wait()


_sc_call = pl.kernel(
    _sc_body,
    out_type=jax.ShapeDtypeStruct((NC, JP, CH), jnp.float32),
    mesh=plsc.VectorSubcoreMesh(
        core_axis_name="c", subcore_axis_name="s",
        num_cores=NC, num_subcores=NS),
    scratch_types=[
        pltpu.VMEM((IROWS, G * K), jnp.int32),
        pltpu.VMEM((NBUF, GN * K, CH), jnp.int32),
        pltpu.VMEM((NBUF, GN, CH), jnp.float32),
        pltpu.VMEM((NBUF, GN, CH), jnp.float32),
        pltpu.VMEM_SHARED((JP, CH), jnp.int32),
    ] + [pltpu.SemaphoreType.DMA] * (2 + 3 * NBUF),
    compiler_params=pltpu.CompilerParams(
        use_tc_tiling_on_sc=False, needs_layout_passes=False),
)

_w = np.arange(C)
_u = _w % CH
_is_v = (_u >= CH // 2).astype(np.int64)
_P = 2 * (_w // CH) + (_u % (CH // 2)) // D
_f_hi = 2 * _P * D + (_w % D) + C * _is_v
_f_lo = _f_hi + D


@jax.jit
def kernel(x, nbr_idx, Wq, Wk, Wv, Wp):
    B, J, Cin = x.shape
    x2 = x.reshape(J, Cin)
    idx = nbr_idx.reshape(J, K).astype(jnp.int32)
    x_pad = jnp.pad(x2, ((0, JP - J), (0, 0)))
    idx3 = jnp.pad(idx, ((0, JP - J), (0, 0))).reshape(NS, IROWS, G * K)

    Wc = jnp.concatenate([Wk.T, Wv.T], axis=1)
    W_hi = Wc[:, _f_hi]
    W_lo = Wc[:, _f_lo]

    q_pair, kv_pair = pl.pallas_call(
        _qkv_body,
        grid=(JP // BLK,),
        in_specs=[
            pl.BlockSpec((BLK, Cin), lambda i: (i, 0)),
            pl.BlockSpec((Cin, C), lambda i: (0, 0)),
            pl.BlockSpec((Cin, C), lambda i: (0, 0)),
            pl.BlockSpec((Cin, C), lambda i: (0, 0)),
        ],
        out_specs=[
            pl.BlockSpec((NC, BLK, CH), lambda i: (0, i, 0)),
            pl.BlockSpec((NC, BLK, CH), lambda i: (0, i, 0)),
        ],
        out_shape=[
            jax.ShapeDtypeStruct((NC, JP, CH), jnp.float32),
            jax.ShapeDtypeStruct((NC, JP, CH), jnp.int32),
        ],
    )(x_pad, Wq.T, W_hi, W_lo)

    out_pair = _sc_call(q_pair, kv_pair, idx3)

    WpT = Wp.T
    y_pad = pl.pallas_call(
        _proj_body,
        grid=(JP // BLK,),
        in_specs=[
            pl.BlockSpec((NC, BLK, CH), lambda i: (0, i, 0)),
            pl.BlockSpec((CH, OUT), lambda i: (0, 0)),
            pl.BlockSpec((CH, OUT), lambda i: (0, 0)),
        ],
        out_specs=pl.BlockSpec((BLK, OUT), lambda i: (i, 0)),
        out_shape=jax.ShapeDtypeStruct((JP, OUT), jnp.float32),
    )(out_pair, WpT[:CH], WpT[CH:])

    return y_pad[:J].reshape(B, J, OUT)


# speedup vs baseline: 1.0453x; 1.0146x over previous
import functools

import numpy as np

import jax
import jax.numpy as jnp
from jax import lax
from jax.experimental import pallas as pl
from jax.experimental.pallas import tpu as pltpu
from jax.experimental.pallas import tpu_sc as plsc

H = 8
D = 16
C = 128
CH = C // 2
OUT = 16
K = 16
L = 16
NC, NS = 2, 16
G = 8
GN = 16
JP = 10240
CHUNK = JP // NS
IROWS = CHUNK // G
ROUNDS = CHUNK // GN
NBUF = 4
BLK = 1024
SCALE = 1.0 / (D ** 0.5)
NEG_SLOPE = 0.2


def _qkv_body(x_ref, wqt_ref, whi_ref, wlo_ref, q_ref, kv_ref):
    xb = x_ref[...]
    q = jnp.dot(xb, wqt_ref[...], preferred_element_type=jnp.float32)
    hi = jnp.dot(xb, whi_ref[...], preferred_element_type=jnp.float32)
    lo = jnp.dot(xb, wlo_ref[...], preferred_element_type=jnp.float32)
    hi_i = lax.convert_element_type(
        lax.bitcast_convert_type(
            lax.convert_element_type(hi, jnp.bfloat16), jnp.int16),
        jnp.int32)
    lo_i = lax.convert_element_type(
        lax.bitcast_convert_type(
            lax.convert_element_type(lo, jnp.bfloat16), jnp.int16),
        jnp.int32)
    word = (hi_i << 16) | (lo_i & 0xFFFF)
    q_ref[0] = q[:, :CH]
    q_ref[1] = q[:, CH:]
    kv_ref[0] = word[:, :CH]
    kv_ref[1] = word[:, CH:]


def _proj_body(o_ref, wpt0_ref, wpt1_ref, y_ref):
    y_ref[...] = (
        jnp.dot(o_ref[0], wpt0_ref[...], preferred_element_type=jnp.float32)
        + jnp.dot(o_ref[1], wpt1_ref[...], preferred_element_type=jnp.float32))


def _sc_body(q_hbm, kv_hbm, idx_hbm, out_hbm, idx_v, kv_v2, q_v2, out_v2,
             kv_sh, isem, ssem, ksem0, ksem1, ksem2, ksem3,
             qsem0, qsem1, qsem2, qsem3, osem0, osem1, osem2, osem3):
    cid = lax.axis_index("c")
    sid = lax.axis_index("s")
    base0 = sid * CHUNK
    iota = lax.iota(jnp.int32, L)
    ksems = (ksem0, ksem1, ksem2, ksem3)
    qsems = (qsem0, qsem1, qsem2, qsem3)
    osems = (osem0, osem1, osem2, osem3)

    pltpu.async_copy(idx_hbm.at[sid], idx_v, isem)

    @pl.when(sid == 0)
    def _():
        pltpu.async_copy(kv_hbm.at[cid], kv_sh, ssem).wait()

    plsc.subcore_barrier()
    pltpu.make_async_copy(idx_hbm.at[sid], idx_v, isem).wait()

    def start_round(r, b):
        base = base0 + r * GN
        pltpu.async_copy(q_hbm.at[cid, pl.ds(base, GN)], q_v2.at[b], qsems[b])
        pltpu.async_copy(kv_sh.at[idx_v.at[2 * r]],
                         kv_v2.at[b, pl.ds(0, G * K)], ksems[b])
        pltpu.async_copy(kv_sh.at[idx_v.at[2 * r + 1]],
                         kv_v2.at[b, pl.ds(G * K, G * K)], ksems[b])

    def wait_round(r, b):
        base = base0 + r * GN
        pltpu.make_async_copy(q_hbm.at[cid, pl.ds(base, GN)], q_v2.at[b],
                              qsems[b]).wait()
        pltpu.make_async_copy(kv_sh.at[idx_v.at[2 * r]],
                              kv_v2.at[b, pl.ds(0, G * K)], ksems[b]).wait()
        pltpu.make_async_copy(kv_sh.at[idx_v.at[2 * r + 1]],
                              kv_v2.at[b, pl.ds(G * K, G * K)],
                              ksems[b]).wait()

    def compute_round(r, b):
        kv_v = kv_v2.at[b]
        q_all = q_v2.at[b]
        out_all = out_v2.at[b]
        base = base0 + r * GN
        row0 = 0

        @plsc.parallel_loop(0, GN, 1, unroll=2)
        def node_body(g):
            g16 = g * L
            row_idx = g16 + iota
            nrow = row0 + g
            mask_hi = jnp.int32(-65536)

            def softmax(acc):
                a = acc * SCALE
                a = jnp.where(a >= 0.0, a, NEG_SLOPE * a)
                m = jnp.max(a)
                e = jnp.exp(a - m)
                return e / jnp.sum(e)

            wvecs = [None] * 4
            for p in range(2):
                q_e = q_all[nrow, pl.ds((2 * p) * D, D)]
                q_o = q_all[nrow, pl.ds((2 * p + 1) * D, D)]
                pe = [jnp.zeros((L,), jnp.float32) for _ in range(2)]
                po = [jnp.zeros((L,), jnp.float32) for _ in range(2)]
                for d in range(D):
                    w_i = p * D + d
                    word = plsc.load_gather(
                        kv_v, [row_idx, jnp.full((L,), w_i, jnp.int32)])
                    fe = plsc.bitcast(word & mask_hi, jnp.float32)
                    fo = plsc.bitcast(word << 16, jnp.float32)
                    pe[d % 2] = pe[d % 2] + q_e[d] * fe
                    po[d % 2] = po[d % 2] + q_o[d] * fo
                wvecs[2 * p] = softmax(pe[0] + pe[1])
                wvecs[2 * p + 1] = softmax(po[0] + po[1])
            for p in range(2):
                we = wvecs[2 * p]
                wo = wvecs[2 * p + 1]
                pe = [jnp.zeros((L,), jnp.float32) for _ in range(2)]
                po = [jnp.zeros((L,), jnp.float32) for _ in range(2)]
                for t in range(L):
                    word = kv_v[g16 + t, pl.ds(CH // 2 + p * D, D)]
                    fe = plsc.bitcast(word & mask_hi, jnp.float32)
                    fo = plsc.bitcast(word << 16, jnp.float32)
                    pe[t % 2] = pe[t % 2] + we[t] * fe
                    po[t % 2] = po[t % 2] + wo[t] * fo
                out_all[nrow, pl.ds((2 * p) * D, D)] = pe[0] + pe[1]
                out_all[nrow, pl.ds((2 * p + 1) * D, D)] = po[0] + po[1]
        pltpu.async_copy(out_all, out_hbm.at[cid, pl.ds(base, GN)], osems[b])

    for b in range(NBUF):
        start_round(b, b)

    def outer(p, carry):
        for b in range(NBUF):
            r = NBUF * p + b
            wait_round(r, b)

            @pl.when(p > 0)
            def _():
                pltpu.make_async_copy(
                    out_v2.at[b], out_hbm.at[cid, pl.ds(base0, GN)],
                    osems[b]).wait()

            compute_round(r, b)

            @pl.when(r + NBUF < ROUNDS)
            def _():
                start_round(r + NBUF, b)
        return carry

    lax.fori_loop(0, ROUNDS // NBUF, outer, 0)
    for b in range(NBUF):
        pltpu.make_async_copy(out_v2.at[b], out_hbm.at[cid, pl.ds(base0, GN)],
                              osems[b]).wait()


_sc_call = pl.kernel(
    _sc_body,
    out_type=jax.ShapeDtypeStruct((NC, JP, CH), jnp.float32),
    mesh=plsc.VectorSubcoreMesh(
        core_axis_name="c", subcore_axis_name="s",
        num_cores=NC, num_subcores=NS),
    scratch_types=[
        pltpu.VMEM((IROWS, G * K), jnp.int32),
        pltpu.VMEM((NBUF, GN * K, CH), jnp.int32),
        pltpu.VMEM((NBUF, GN, CH), jnp.float32),
        pltpu.VMEM((NBUF, GN, CH), jnp.float32),
        pltpu.VMEM_SHARED((JP, CH), jnp.int32),
    ] + [pltpu.SemaphoreType.DMA] * (2 + 3 * NBUF),
    compiler_params=pltpu.CompilerParams(
        use_tc_tiling_on_sc=False, needs_layout_passes=False),
)

_w = np.arange(C)
_u = _w % CH
_is_v = (_u >= CH // 2).astype(np.int64)
_P = 2 * (_w // CH) + (_u % (CH // 2)) // D
_f_hi = 2 * _P * D + (_w % D) + C * _is_v
_f_lo = _f_hi + D


@jax.jit
def kernel(x, nbr_idx, Wq, Wk, Wv, Wp):
    B, J, Cin = x.shape
    x2 = x.reshape(J, Cin)
    idx = nbr_idx.reshape(J, K).astype(jnp.int32)
    idx3 = jnp.pad(idx, ((0, JP - J), (0, 0))).reshape(NS, IROWS, G * K)

    Wc = jnp.concatenate([Wk.T, Wv.T], axis=1)
    W_hi = Wc[:, _f_hi]
    W_lo = Wc[:, _f_lo]

    q_pair, kv_pair = pl.pallas_call(
        _qkv_body,
        grid=(JP // BLK,),
        in_specs=[
            pl.BlockSpec((BLK, Cin), lambda i: (i, 0)),
            pl.BlockSpec((Cin, C), lambda i: (0, 0)),
            pl.BlockSpec((Cin, C), lambda i: (0, 0)),
            pl.BlockSpec((Cin, C), lambda i: (0, 0)),
        ],
        out_specs=[
            pl.BlockSpec((NC, BLK, CH), lambda i: (0, i, 0)),
            pl.BlockSpec((NC, BLK, CH), lambda i: (0, i, 0)),
        ],
        out_shape=[
            jax.ShapeDtypeStruct((NC, JP, CH), jnp.float32),
            jax.ShapeDtypeStruct((NC, JP, CH), jnp.int32),
        ],
    )(x2, Wq.T, W_hi, W_lo)

    out_pair = _sc_call(q_pair, kv_pair, idx3)

    WpT = Wp.T
    y_pad = pl.pallas_call(
        _proj_body,
        grid=(JP // BLK,),
        in_specs=[
            pl.BlockSpec((NC, BLK, CH), lambda i: (0, i, 0)),
            pl.BlockSpec((CH, OUT), lambda i: (0, 0)),
            pl.BlockSpec((CH, OUT), lambda i: (0, 0)),
        ],
        out_specs=pl.BlockSpec((BLK, OUT), lambda i: (i, 0)),
        out_shape=jax.ShapeDtypeStruct((JP, OUT), jnp.float32),
    )(out_pair, WpT[:CH], WpT[CH:])

    return y_pad[:J].reshape(B, J, OUT)
